# Initial kernel scaffold; baseline (speedup 1.0000x reference)
import jax
import jax.numpy as jnp
from jax.experimental import pallas as pl


def kernel(x, W_in, Wd, b, dendrites):
    raise NotImplementedError("write your pallas kernel here")


# speedup vs baseline: 35.1147x; 35.1147x over previous
import jax
import jax.numpy as jnp
from jax import lax
from jax.experimental import pallas as pl
from jax.experimental.pallas import tpu as pltpu
from jax.experimental.pallas import tpu_sc as plsc

_LANES = 16
_NC = 2
_NS = 16
_NW = _NC * _NS


def _weff_sparsecore(W_in_flat, d_flat, Wd_flat, U, C, SEQL):
    total = U * C
    per_w = total // _NW
    rows_per_w = U // _NW
    row_chunks = C // _LANES

    mesh = plsc.VectorSubcoreMesh(core_axis_name="c", subcore_axis_name="s")

    def body(win_hbm, d_hbm, wd_hbm, weff_hbm, wd_v, win_v, d_v, eff_v):
        wid = lax.axis_index("s") * _NC + lax.axis_index("c")
        base = wid * per_w
        base_u = wid * rows_per_w
        pltpu.sync_copy(wd_hbm, wd_v)
        pltpu.sync_copy(win_hbm.at[pl.ds(base, per_w)], win_v)
        pltpu.sync_copy(d_hbm.at[pl.ds(base, per_w)], d_v)
        for r in range(rows_per_w):
            u = base_u + r

            def chunk(j, carry, r=r, u=u):
                off = r * C + j * _LANES
                d = d_v[pl.ds(off, _LANES)]
                g = plsc.load_gather(wd_v, [d * U + u])
                w = win_v[pl.ds(off, _LANES)]
                eff_v[pl.ds(off, _LANES)] = jnp.maximum(w, 5e-5) * g
                return carry

            lax.fori_loop(0, row_chunks, chunk, 0)
        pltpu.sync_copy(eff_v, weff_hbm.at[pl.ds(base, per_w)])

    return pl.kernel(
        body,
        out_type=jax.ShapeDtypeStruct((total,), jnp.float32),
        mesh=mesh,
        compiler_params=pltpu.CompilerParams(needs_layout_passes=False),
        scratch_types=[
            pltpu.VMEM((SEQL * U,), jnp.float32),
            pltpu.VMEM((per_w,), jnp.float32),
            pltpu.VMEM((per_w,), jnp.int32),
            pltpu.VMEM((per_w,), jnp.float32),
        ],
    )(W_in_flat, d_flat, Wd_flat)


def _mm_body(x_ref, w_ref, b_ref, o_ref):
    acc = lax.dot_general(
        x_ref[...], w_ref[...], (((1,), (1,)), ((), ())),
        preferred_element_type=jnp.float32,
        precision=lax.Precision.HIGHEST,
    )
    o_ref[...] = jnp.maximum(acc + b_ref[...], 0.0)


def kernel(x, W_in, Wd, b, dendrites):
    B, C = x.shape
    U = W_in.shape[0]
    SEQL = Wd.shape[0]

    weff = _weff_sparsecore(
        W_in.reshape(-1), dendrites.reshape(-1), Wd.reshape(-1), U, C, SEQL
    ).reshape(U, C)

    return pl.pallas_call(
        _mm_body,
        out_shape=jax.ShapeDtypeStruct((B, U), jnp.float32),
    )(x, weff, b.reshape(1, U))


# speedup vs baseline: 48.0528x; 1.3685x over previous
import jax
import jax.numpy as jnp
from jax import lax
from jax.experimental import pallas as pl
from jax.experimental.pallas import tpu as pltpu
from jax.experimental.pallas import tpu_sc as plsc

_LANES = 16
_NC = 2
_NS = 16
_NW = _NC * _NS


def _weff_sparsecore(W_in_flat, d_flat, WdT_flat, U, C, SEQL):
    total = U * C
    per_w = total // _NW
    rows_per_w = U // _NW

    mesh = plsc.VectorSubcoreMesh(core_axis_name="c", subcore_axis_name="s")

    def body(win_hbm, d_hbm, wdt_hbm, weff_hbm, wdt_v, win_v, d_v, eff_v):
        wid = lax.axis_index("s") * _NC + lax.axis_index("c")
        base = wid * per_w
        base_u = wid * rows_per_w
        pltpu.sync_copy(wdt_hbm.at[pl.ds(base_u * SEQL, rows_per_w * SEQL)],
                        wdt_v)
        pltpu.sync_copy(win_hbm.at[pl.ds(base, per_w)], win_v)
        pltpu.sync_copy(d_hbm.at[pl.ds(base, per_w)], d_v)
        for r in range(rows_per_w):
            rbase = r * C
            roff = r * SEQL

            @plsc.parallel_loop(0, C, step=_LANES, unroll=4)
            def _(i, rbase=rbase, roff=roff):
                off = rbase + i
                d = d_v[pl.ds(off, _LANES)]
                g = plsc.load_gather(wdt_v, [d + roff])
                w = win_v[pl.ds(off, _LANES)]
                eff_v[pl.ds(off, _LANES)] = jnp.maximum(w, 5e-5) * g

        pltpu.sync_copy(eff_v, weff_hbm.at[pl.ds(base, per_w)])

    return pl.kernel(
        body,
        out_type=jax.ShapeDtypeStruct((total,), jnp.float32),
        mesh=mesh,
        compiler_params=pltpu.CompilerParams(needs_layout_passes=False),
        scratch_types=[
            pltpu.VMEM((rows_per_w * SEQL,), jnp.float32),
            pltpu.VMEM((per_w,), jnp.float32),
            pltpu.VMEM((per_w,), jnp.int32),
            pltpu.VMEM((per_w,), jnp.float32),
        ],
    )(W_in_flat, d_flat, WdT_flat)


def _mm_body(x_ref, w_ref, b_ref, o_ref):
    acc = lax.dot_general(
        x_ref[...], w_ref[...], (((1,), (1,)), ((), ())),
        preferred_element_type=jnp.float32,
    )
    o_ref[...] = jnp.maximum(acc + b_ref[...], 0.0)


def kernel(x, W_in, Wd, b, dendrites):
    B, C = x.shape
    U = W_in.shape[0]
    SEQL = Wd.shape[0]

    weff = _weff_sparsecore(
        W_in.reshape(-1), dendrites.reshape(-1), Wd.T.reshape(-1), U, C, SEQL
    ).reshape(U, C)

    return pl.pallas_call(
        _mm_body,
        out_shape=jax.ShapeDtypeStruct((B, U), jnp.float32),
    )(x, weff, b.reshape(1, U))
